# baseline (speedup 1.0000x reference)
import functools

import jax
import jax.numpy as jnp
from jax import lax
from jax.experimental import pallas as pl
from jax.experimental.pallas import tpu as pltpu
from jax.experimental.pallas import tpu_sc as plsc

_EMBED = 16
_BATCH = 16384
_ROWS = 128
_COLS = 128
_NC, _NS = 2, 16
_NW = _NC * _NS
_RPW = _ROWS // _NW
_REGS = 1e-05

_mesh = plsc.VectorSubcoreMesh(core_axis_name="c", subcore_axis_name="s")


@functools.partial(
    pl.kernel,
    mesh=_mesh,
    compiler_params=pltpu.CompilerParams(use_tc_tiling_on_sc=False),
    out_type=[
        jax.ShapeDtypeStruct((_ROWS, _COLS, _EMBED), jnp.float32),
        jax.ShapeDtypeStruct((_ROWS, _COLS, _EMBED), jnp.float32),
        jax.ShapeDtypeStruct((_ROWS, _COLS, _EMBED), jnp.float32),
    ],
    scratch_types=[
        pltpu.VMEM((_RPW, _COLS), jnp.int32),
        pltpu.VMEM((_RPW, _COLS), jnp.int32),
        pltpu.VMEM((_RPW, _COLS), jnp.int32),
        pltpu.VMEM((_RPW, _COLS, _EMBED), jnp.float32),
        pltpu.VMEM((_RPW, _COLS, _EMBED), jnp.float32),
        pltpu.VMEM((_RPW, _COLS, _EMBED), jnp.float32),
        pltpu.SemaphoreType.DMA,
    ],
)
def _sc_gather(user_h, pos_h, neg_h, ue_h, ie_h, out_u, out_p, out_n,
               idx_u, idx_p, idx_n, buf_u, buf_p, buf_n, sem):
    wid = lax.axis_index("s") * _NC + lax.axis_index("c")
    base = wid * _RPW
    pltpu.sync_copy(user_h.at[pl.ds(base, _RPW)], idx_u)
    pltpu.sync_copy(pos_h.at[pl.ds(base, _RPW)], idx_p)
    pltpu.sync_copy(neg_h.at[pl.ds(base, _RPW)], idx_n)
    copies = []
    for j in range(_RPW):
        copies.append(pltpu.async_copy(ue_h.at[idx_u.at[j]], buf_u.at[j], sem))
        copies.append(pltpu.async_copy(ie_h.at[idx_p.at[j]], buf_p.at[j], sem))
        copies.append(pltpu.async_copy(ie_h.at[idx_n.at[j]], buf_n.at[j], sem))
    for c in copies:
        c.wait()
    pltpu.sync_copy(buf_u, out_u.at[pl.ds(base, _RPW)])
    pltpu.sync_copy(buf_p, out_p.at[pl.ds(base, _RPW)])
    pltpu.sync_copy(buf_n, out_n.at[pl.ds(base, _RPW)])


def _tc_loss(u_ref, p_ref, n_ref, bpr_ref, reg_ref):
    u = u_ref[...]
    p = p_ref[...]
    n = n_ref[...]
    sumsq = jnp.sum(u * u) + jnp.sum(p * p) + jnp.sum(n * n)
    x = (p - n) * u
    ri = lax.broadcasted_iota(jnp.int32, (128, 128 // _EMBED), 0)
    cj = lax.broadcasted_iota(jnp.int32, (128, 128 // _EMBED), 1)
    m = jnp.where(ri // _EMBED == cj, 1.0, 0.0).astype(jnp.float32)
    d = jax.lax.dot(x, m, precision=jax.lax.Precision.HIGHEST)
    sp = jnp.maximum(-d, 0.0) + jnp.log(1.0 + jnp.exp(-jnp.abs(d)))
    bpr_ref[...] = (jnp.sum(sp) / jnp.float32(_BATCH)).reshape(1, 1)
    reg_ref[...] = (jnp.float32(_REGS * 0.5) * sumsq).reshape(1, 1)


def kernel(user, pos, neg, user_embedding, item_embedding):
    user2 = user.reshape(_ROWS, _COLS).astype(jnp.int32)
    pos2 = pos.reshape(_ROWS, _COLS).astype(jnp.int32)
    neg2 = neg.reshape(_ROWS, _COLS).astype(jnp.int32)
    gu, gp, gn = _sc_gather(user2, pos2, neg2, user_embedding, item_embedding)
    flat = (_BATCH * _EMBED // 128, 128)
    bpr, reg = pl.pallas_call(
        _tc_loss,
        out_shape=(
            jax.ShapeDtypeStruct((1, 1), jnp.float32),
            jax.ShapeDtypeStruct((1, 1), jnp.float32),
        ),
    )(gu.reshape(flat), gp.reshape(flat), gn.reshape(flat))
    return (bpr[0, 0], reg[0, 0])
